# baseline (speedup 1.0000x reference)
import jax
import jax.numpy as jnp
from jax import lax
from jax.experimental import pallas as pl
from jax.experimental.pallas import tpu as pltpu
from jax.experimental.pallas import tpu_sc as plsc

F = 2048
T = 32
NV = 32
NCROPS = 10
B = NV * NCROPS
NSIZE = 16
K = 3
S = 8
ROWS = S * T
NW = 32
RPW = 30

_BF = jnp.bfloat16
_F32 = jnp.float32


def _abc_body(x_ref, w1_ref, b1_ref, w2_ref, b2_ref, w3_ref, b3_ref, w4_ref,
              gw_ref, gb_ref, thw_ref, thb_ref, phw_ref, phb_ref,
              ww_ref, wb_ref, w5a_ref, w5b_ref, b5_ref,
              feat_ref, norms_ref):
    xb = x_ref[...].astype(_BF)
    zpad = jnp.zeros((S, 4, F), _BF)
    xp = jnp.concatenate([zpad, xb, zpad], axis=1)
    x = xb.reshape(ROWS, F)

    outs = []
    for j, (w_ref, b_ref, d) in enumerate(
            ((w1_ref, b1_ref, 1), (w2_ref, b2_ref, 2), (w3_ref, b3_ref, 4))):
        acc = jnp.broadcast_to(b_ref[...][None, :], (ROWS, 512))
        for k in range(3):
            sl = xp[:, 4 - d + d * k:4 - d + d * k + T, :].reshape(ROWS, F)
            acc = acc + jnp.dot(sl, w_ref[k], preferred_element_type=_F32)
        outs.append(jnp.maximum(acc, 0.0).astype(_BF))

    o4 = jnp.maximum(jnp.dot(x, w4_ref[...], preferred_element_type=_F32), 0.0)
    o4b = o4.astype(_BF)

    gx = jnp.dot(o4b, gw_ref[...], preferred_element_type=_F32) + gb_ref[...][None, :]
    th = jnp.dot(o4b, thw_ref[...], preferred_element_type=_F32) + thb_ref[...][None, :]
    ph = jnp.dot(o4b, phw_ref[...], preferred_element_type=_F32) + phb_ref[...][None, :]
    gxb, thb, phb = gx.astype(_BF), th.astype(_BF), ph.astype(_BF)
    logits = lax.dot_general(thb, phb, (((1,), (1,)), ((), ())),
                             preferred_element_type=_F32)
    ri = lax.broadcasted_iota(jnp.int32, (ROWS, ROWS), 0)
    ci = lax.broadcasted_iota(jnp.int32, (ROWS, ROWS), 1)
    same = (ri // T) == (ci // T)
    logits = jnp.where(same, logits, jnp.float32(-3e38))
    att = jax.nn.softmax(logits, axis=-1).astype(_BF)
    y = jnp.dot(att, gxb, preferred_element_type=_F32).astype(_BF)
    z = (jnp.dot(y, ww_ref[...], preferred_element_type=_F32)
         + wb_ref[...][None, :] + o4)
    zb = z.astype(_BF).reshape(S, T, 512)

    od = jnp.concatenate(outs, axis=1).reshape(S, T, 1536)
    odp = jnp.concatenate(
        [jnp.zeros((S, 1, 1536), _BF), od, jnp.zeros((S, 1, 1536), _BF)], axis=1)
    zp = jnp.concatenate(
        [jnp.zeros((S, 1, 512), _BF), zb, jnp.zeros((S, 1, 512), _BF)], axis=1)
    acc = jnp.broadcast_to(b5_ref[...][None, :], (ROWS, F))
    for k in range(3):
        sa = odp[:, k:k + T, :].reshape(ROWS, 1536)
        sb = zp[:, k:k + T, :].reshape(ROWS, 512)
        acc = (acc + jnp.dot(sa, w5a_ref[k], preferred_element_type=_F32)
               + jnp.dot(sb, w5b_ref[k], preferred_element_type=_F32))
    feat = jnp.maximum(acc, 0.0) + x_ref[...].reshape(ROWS, F)
    feat_ref[...] = feat.reshape(S, T, F)
    norms_ref[...] = jnp.sqrt(jnp.sum(feat * feat, axis=1)).reshape(1, S, T)


def _e_body(feat_ref, f1_ref, f1b_ref, f2_ref, f2b_ref, f3_ref, f3b_ref,
            scores_ref):
    feat = feat_ref[...].reshape(ROWS, F)
    h1 = jnp.maximum(jnp.dot(feat.astype(_BF), f1_ref[...],
                             preferred_element_type=_F32)
                     + f1b_ref[...][None, :], 0.0)
    h2 = jnp.maximum(jnp.dot(h1.astype(_BF), f2_ref[...],
                             preferred_element_type=_F32)
                     + f2b_ref[...][None, :], 0.0)
    logit = jnp.sum(h2 * f3_ref[...][None, :], axis=1) + f3b_ref[0]
    scores_ref[...] = jax.nn.sigmoid(logit).reshape(1, S, T)


def _top3(a):
    col = lax.broadcasted_iota(jnp.int32, (NSIZE, T), 1)
    idxs, onehots = [], []
    for _ in range(K):
        mx = jnp.max(a, axis=1, keepdims=True)
        cand = jnp.where(a >= mx, col, jnp.int32(2 * T))
        idx = jnp.min(cand, axis=1)
        oh = (col == idx[:, None])
        idxs.append(idx)
        onehots.append(oh)
        a = jnp.where(oh, jnp.float32(-3e38), a)
    return idxs, onehots


def _d_body(norms_ref, scores_ref, fm_ref, x_ref, sab_ref, sno_ref, idx_ref):
    norms = norms_ref[...]
    scores = scores_ref[...]
    srow = lax.broadcasted_iota(jnp.int32, (NV, B), 1)
    vrow = lax.broadcasted_iota(jnp.int32, (NV, B), 0)
    sel = jnp.where(srow // NCROPS == vrow, jnp.float32(1.0 / NCROPS),
                    jnp.float32(0.0))
    fm = jnp.dot(sel, norms, preferred_element_type=_F32,
                 precision=lax.Precision.HIGHEST)
    x2 = jnp.dot(sel, scores, preferred_element_type=_F32,
                 precision=lax.Precision.HIGHEST)
    fm_ref[...] = fm
    x_ref[...] = x2

    idx_abn, oh_abn = _top3(fm[NSIZE:, :])
    idx_nor, oh_nor = _top3(fm[:NSIZE, :])
    xa = x2[NSIZE:, :]
    xn = x2[:NSIZE, :]
    sab = sum(jnp.sum(jnp.where(oh, xa, 0.0), axis=1) for oh in oh_abn) / K
    sno = sum(jnp.sum(jnp.where(oh, xn, 0.0), axis=1) for oh in oh_nor) / K
    sab_ref[...] = sab.reshape(NSIZE, 1)
    sno_ref[...] = sno.reshape(NSIZE, 1)

    w = lax.broadcasted_iota(jnp.int32, (NW, T), 0)
    i = lax.broadcasted_iota(jnp.int32, (NW, T), 1)
    j = w * RPW + i
    r = j % (NSIZE * K)
    n = r // K
    c = (j % 480) // (NSIZE * K)
    abn = j < 480
    t = jnp.zeros((NW, T), jnp.int32)
    for m in range(NSIZE * K):
        tm = jnp.where(abn, idx_abn[m % K][m // K], idx_nor[m % K][m // K])
        t = jnp.where(r == m, tm, t)
    sample = jnp.where(abn, 160, 0) + n * NCROPS + c
    flat = sample * T + t
    idx_ref[...] = jnp.where((i < RPW) & (j < 960), flat, 0)


def _sc_gather_body(table_ref, idx_ref, out_ref, idx_v, rows_v, sem):
    wid = lax.axis_index("s") * 2 + lax.axis_index("c")
    pltpu.sync_copy(idx_ref.at[wid], idx_v)
    pltpu.async_copy(table_ref.at[idx_v], rows_v, sem).wait()
    pltpu.sync_copy(rows_v, out_ref.at[pl.ds(wid * T, T)])


def _sc_gather(table, idx2d):
    mesh = plsc.VectorSubcoreMesh(core_axis_name="c", subcore_axis_name="s")
    return pl.kernel(
        _sc_gather_body,
        out_type=jax.ShapeDtypeStruct((NW * T, F), jnp.float32),
        mesh=mesh,
        scratch_types=[
            pltpu.VMEM((T,), jnp.int32),
            pltpu.VMEM((T, F), jnp.float32),
            pltpu.SemaphoreType.DMA,
        ],
    )(table, idx2d)


def kernel(inputs, c1_w, c1_b, c2_w, c2_b, c3_w, c3_b, c4_w, c5_w, c5_b,
           g_w, g_b, th_w, th_b, ph_w, ph_b, W_w, W_b,
           fc1_w, fc1_b, fc2_w, fc2_b, fc3_w, fc3_b):
    x = inputs.reshape(B, T, F)

    w1 = jnp.transpose(c1_w, (2, 1, 0)).astype(_BF)
    w2 = jnp.transpose(c2_w, (2, 1, 0)).astype(_BF)
    w3 = jnp.transpose(c3_w, (2, 1, 0)).astype(_BF)
    w4 = jnp.transpose(c4_w[:, :, 0], (1, 0)).astype(_BF)
    gw = jnp.transpose(g_w[:, :, 0], (1, 0)).astype(_BF)
    thw = jnp.transpose(th_w[:, :, 0], (1, 0)).astype(_BF)
    phw = jnp.transpose(ph_w[:, :, 0], (1, 0)).astype(_BF)
    ww = jnp.transpose(W_w[:, :, 0], (1, 0)).astype(_BF)
    w5 = jnp.transpose(c5_w, (2, 1, 0)).astype(_BF)
    w5a = w5[:, :1536, :]
    w5b = w5[:, 1536:, :]
    f1 = jnp.transpose(fc1_w, (1, 0)).astype(_BF)
    f2 = jnp.transpose(fc2_w, (1, 0)).astype(_BF)
    f3 = fc3_w[0]

    nchunks = B // S
    full = lambda shp: pl.BlockSpec(shp, lambda i: (0,) * len(shp))
    feat, norms = pl.pallas_call(
        _abc_body,
        grid=(nchunks,),
        in_specs=[
            pl.BlockSpec((S, T, F), lambda i: (i, 0, 0)),
            full((3, F, 512)), full((512,)),
            full((3, F, 512)), full((512,)),
            full((3, F, 512)), full((512,)),
            full((F, 512)),
            full((512, 256)), full((256,)),
            full((512, 256)), full((256,)),
            full((512, 256)), full((256,)),
            full((256, 512)), full((512,)),
            full((3, 1536, F)), full((3, 512, F)), full((F,)),
        ],
        out_specs=[
            pl.BlockSpec((S, T, F), lambda i: (i, 0, 0)),
            pl.BlockSpec((1, S, T), lambda i: (i, 0, 0)),
        ],
        out_shape=[
            jax.ShapeDtypeStruct((B, T, F), jnp.float32),
            jax.ShapeDtypeStruct((nchunks, S, T), jnp.float32),
        ],
    )(x, w1, c1_b, w2, c2_b, w3, c3_b, w4,
      gw, g_b, thw, th_b, phw, ph_b, ww, W_b,
      w5a, w5b, c5_b)

    scores = pl.pallas_call(
        _e_body,
        grid=(nchunks,),
        in_specs=[
            pl.BlockSpec((S, T, F), lambda i: (i, 0, 0)),
            full((F, 512)), full((512,)),
            full((512, 128)), full((128,)),
            full((128,)), full((1,)),
        ],
        out_specs=[pl.BlockSpec((1, S, T), lambda i: (i, 0, 0))],
        out_shape=[jax.ShapeDtypeStruct((nchunks, S, T), jnp.float32)],
    )(feat, f1, fc1_b, f2, fc2_b, f3, fc3_b)[0]
    norms = norms.reshape(B, T)
    scores = scores.reshape(B, T)

    fm, x2, sab, sno, idx2d = pl.pallas_call(
        _d_body,
        grid=(1,),
        in_specs=[full((B, T)), full((B, T))],
        out_specs=[full((NV, T)), full((NV, T)),
                   full((NSIZE, 1)), full((NSIZE, 1)), full((NW, T))],
        out_shape=[
            jax.ShapeDtypeStruct((NV, T), jnp.float32),
            jax.ShapeDtypeStruct((NV, T), jnp.float32),
            jax.ShapeDtypeStruct((NSIZE, 1), jnp.float32),
            jax.ShapeDtypeStruct((NSIZE, 1), jnp.float32),
            jax.ShapeDtypeStruct((NW, T), jnp.int32),
        ],
    )(norms, scores)

    buf = _sc_gather(feat.reshape(B * T, F), idx2d)
    buf = buf.reshape(NW, T, F)[:, :RPW, :].reshape(NW * RPW, F)
    fsa = buf[:480].reshape(160, K, F)
    fsn = buf[480:].reshape(160, K, F)
    xout = x2[:, :, None]
    return (sab, sno, fsa, fsn, fsa, fsa, xout, fsa, fsa, fm)


# speedup vs baseline: 1.0274x; 1.0274x over previous
import jax
import jax.numpy as jnp
from jax import lax
from jax.experimental import pallas as pl
from jax.experimental.pallas import tpu as pltpu
from jax.experimental.pallas import tpu_sc as plsc

F = 2048
T = 32
NV = 32
NCROPS = 10
B = NV * NCROPS
NSIZE = 16
K = 3
S = 8
ROWS = S * T
NW = 32
RPW = 30

_BF = jnp.bfloat16
_F32 = jnp.float32


def _ab_body(x_ref, w1_ref, b1_ref, w2_ref, b2_ref, w3_ref, b3_ref, w4_ref,
             gw_ref, gb_ref, thw_ref, thb_ref, phw_ref, phb_ref,
             ww_ref, wb_ref, outd_ref, z_ref):
    xb = x_ref[...].astype(_BF)
    zpad = jnp.zeros((S, 4, F), _BF)
    xp = jnp.concatenate([zpad, xb, zpad], axis=1)
    x = xb.reshape(ROWS, F)

    for j, (w_ref, b_ref, d) in enumerate(
            ((w1_ref, b1_ref, 1), (w2_ref, b2_ref, 2), (w3_ref, b3_ref, 4))):
        acc = jnp.broadcast_to(b_ref[...][None, :], (ROWS, 512))
        for k in range(3):
            sl = xp[:, 4 - d + d * k:4 - d + d * k + T, :].reshape(ROWS, F)
            acc = acc + jnp.dot(sl, w_ref[k], preferred_element_type=_F32)
        outd_ref[:, :, j * 512:(j + 1) * 512] = (
            jnp.maximum(acc, 0.0).astype(_BF).reshape(S, T, 512))

    o4 = jnp.maximum(jnp.dot(x, w4_ref[...], preferred_element_type=_F32), 0.0)
    o4b = o4.astype(_BF)

    gx = jnp.dot(o4b, gw_ref[...], preferred_element_type=_F32) + gb_ref[...][None, :]
    th = jnp.dot(o4b, thw_ref[...], preferred_element_type=_F32) + thb_ref[...][None, :]
    ph = jnp.dot(o4b, phw_ref[...], preferred_element_type=_F32) + phb_ref[...][None, :]
    gxb, thb, phb = gx.astype(_BF), th.astype(_BF), ph.astype(_BF)
    logits = lax.dot_general(thb, phb, (((1,), (1,)), ((), ())),
                             preferred_element_type=_F32)
    ri = lax.broadcasted_iota(jnp.int32, (ROWS, ROWS), 0)
    ci = lax.broadcasted_iota(jnp.int32, (ROWS, ROWS), 1)
    same = (ri // T) == (ci // T)
    logits = jnp.where(same, logits, jnp.float32(-3e38))
    att = jax.nn.softmax(logits, axis=-1).astype(_BF)
    y = jnp.dot(att, gxb, preferred_element_type=_F32).astype(_BF)
    z = (jnp.dot(y, ww_ref[...], preferred_element_type=_F32)
         + wb_ref[...][None, :] + o4)
    z_ref[...] = z.astype(_BF).reshape(S, T, 512)


def _c_body(od_ref, z_ref, x_ref, w5a_ref, w5b_ref, b5_ref,
            f1_ref, f1b_ref, f2_ref, f2b_ref, f3_ref, f3b_ref,
            feat_ref, norms_ref, scores_ref):
    od = od_ref[...]
    zz = z_ref[...]
    odp = jnp.concatenate(
        [jnp.zeros((S, 1, 1536), _BF), od, jnp.zeros((S, 1, 1536), _BF)], axis=1)
    zp = jnp.concatenate(
        [jnp.zeros((S, 1, 512), _BF), zz, jnp.zeros((S, 1, 512), _BF)], axis=1)
    acc = jnp.broadcast_to(b5_ref[...][None, :], (ROWS, F))
    for k in range(3):
        sa = odp[:, k:k + T, :].reshape(ROWS, 1536)
        sb = zp[:, k:k + T, :].reshape(ROWS, 512)
        acc = (acc + jnp.dot(sa, w5a_ref[k], preferred_element_type=_F32)
               + jnp.dot(sb, w5b_ref[k], preferred_element_type=_F32))
    feat = jnp.maximum(acc, 0.0) + x_ref[...].reshape(ROWS, F)
    feat_ref[...] = feat.reshape(S, T, F)
    norms_ref[...] = jnp.sqrt(jnp.sum(feat * feat, axis=1)).reshape(1, S, T)
    h1 = jnp.maximum(jnp.dot(feat.astype(_BF), f1_ref[...],
                             preferred_element_type=_F32)
                     + f1b_ref[...][None, :], 0.0)
    h2 = jnp.maximum(jnp.dot(h1.astype(_BF), f2_ref[...],
                             preferred_element_type=_F32)
                     + f2b_ref[...][None, :], 0.0)
    logit = jnp.sum(h2 * f3_ref[...][None, :], axis=1) + f3b_ref[0]
    scores_ref[...] = jax.nn.sigmoid(logit).reshape(1, S, T)


def _top3(a):
    col = lax.broadcasted_iota(jnp.int32, (NSIZE, T), 1)
    idxs, onehots = [], []
    for _ in range(K):
        mx = jnp.max(a, axis=1, keepdims=True)
        cand = jnp.where(a >= mx, col, jnp.int32(2 * T))
        idx = jnp.min(cand, axis=1)
        oh = (col == idx[:, None])
        idxs.append(idx)
        onehots.append(oh)
        a = jnp.where(oh, jnp.float32(-3e38), a)
    return idxs, onehots


def _d_body(norms_ref, scores_ref, fm_ref, x_ref, sab_ref, sno_ref, idx_ref):
    norms = norms_ref[...]
    scores = scores_ref[...]
    srow = lax.broadcasted_iota(jnp.int32, (NV, B), 1)
    vrow = lax.broadcasted_iota(jnp.int32, (NV, B), 0)
    sel = jnp.where(srow // NCROPS == vrow, jnp.float32(1.0 / NCROPS),
                    jnp.float32(0.0))
    fm = jnp.dot(sel, norms, preferred_element_type=_F32,
                 precision=lax.Precision.HIGHEST)
    x2 = jnp.dot(sel, scores, preferred_element_type=_F32,
                 precision=lax.Precision.HIGHEST)
    fm_ref[...] = fm
    x_ref[...] = x2

    idx_abn, oh_abn = _top3(fm[NSIZE:, :])
    idx_nor, oh_nor = _top3(fm[:NSIZE, :])
    xa = x2[NSIZE:, :]
    xn = x2[:NSIZE, :]
    sab = sum(jnp.sum(jnp.where(oh, xa, 0.0), axis=1) for oh in oh_abn) / K
    sno = sum(jnp.sum(jnp.where(oh, xn, 0.0), axis=1) for oh in oh_nor) / K
    sab_ref[...] = sab.reshape(NSIZE, 1)
    sno_ref[...] = sno.reshape(NSIZE, 1)

    w = lax.broadcasted_iota(jnp.int32, (NW, T), 0)
    i = lax.broadcasted_iota(jnp.int32, (NW, T), 1)
    j = w * RPW + i
    r = j % (NSIZE * K)
    n = r // K
    c = (j % 480) // (NSIZE * K)
    abn = j < 480
    t = jnp.zeros((NW, T), jnp.int32)
    for m in range(NSIZE * K):
        tm = jnp.where(abn, idx_abn[m % K][m // K], idx_nor[m % K][m // K])
        t = jnp.where(r == m, tm, t)
    sample = jnp.where(abn, 160, 0) + n * NCROPS + c
    flat = sample * T + t
    idx_ref[...] = jnp.where((i < RPW) & (j < 960), flat, 0)


def _sc_gather_body(table_ref, idx_ref, out_ref, idx_v, rows_v, sem):
    wid = lax.axis_index("s") * 2 + lax.axis_index("c")
    pltpu.sync_copy(idx_ref.at[wid], idx_v)
    pltpu.async_copy(table_ref.at[idx_v], rows_v, sem).wait()
    pltpu.sync_copy(rows_v, out_ref.at[pl.ds(wid * T, T)])


def _sc_gather(table, idx2d):
    mesh = plsc.VectorSubcoreMesh(core_axis_name="c", subcore_axis_name="s")
    return pl.kernel(
        _sc_gather_body,
        out_type=jax.ShapeDtypeStruct((NW * T, F), jnp.float32),
        mesh=mesh,
        scratch_types=[
            pltpu.VMEM((T,), jnp.int32),
            pltpu.VMEM((T, F), jnp.float32),
            pltpu.SemaphoreType.DMA,
        ],
    )(table, idx2d)


def kernel(inputs, c1_w, c1_b, c2_w, c2_b, c3_w, c3_b, c4_w, c5_w, c5_b,
           g_w, g_b, th_w, th_b, ph_w, ph_b, W_w, W_b,
           fc1_w, fc1_b, fc2_w, fc2_b, fc3_w, fc3_b):
    x = inputs.reshape(B, T, F)

    w1 = jnp.transpose(c1_w, (2, 1, 0)).astype(_BF)
    w2 = jnp.transpose(c2_w, (2, 1, 0)).astype(_BF)
    w3 = jnp.transpose(c3_w, (2, 1, 0)).astype(_BF)
    w4 = jnp.transpose(c4_w[:, :, 0], (1, 0)).astype(_BF)
    gw = jnp.transpose(g_w[:, :, 0], (1, 0)).astype(_BF)
    thw = jnp.transpose(th_w[:, :, 0], (1, 0)).astype(_BF)
    phw = jnp.transpose(ph_w[:, :, 0], (1, 0)).astype(_BF)
    ww = jnp.transpose(W_w[:, :, 0], (1, 0)).astype(_BF)
    w5 = jnp.transpose(c5_w, (2, 1, 0)).astype(_BF)
    w5a = w5[:, :1536, :]
    w5b = w5[:, 1536:, :]
    f1 = jnp.transpose(fc1_w, (1, 0)).astype(_BF)
    f2 = jnp.transpose(fc2_w, (1, 0)).astype(_BF)
    f3 = fc3_w[0]

    nchunks = B // S
    full = lambda shp: pl.BlockSpec(shp, lambda i: (0,) * len(shp))
    outd, z = pl.pallas_call(
        _ab_body,
        grid=(nchunks,),
        in_specs=[
            pl.BlockSpec((S, T, F), lambda i: (i, 0, 0)),
            full((3, F, 512)), full((512,)),
            full((3, F, 512)), full((512,)),
            full((3, F, 512)), full((512,)),
            full((F, 512)),
            full((512, 256)), full((256,)),
            full((512, 256)), full((256,)),
            full((512, 256)), full((256,)),
            full((256, 512)), full((512,)),
        ],
        out_specs=[
            pl.BlockSpec((S, T, 1536), lambda i: (i, 0, 0)),
            pl.BlockSpec((S, T, 512), lambda i: (i, 0, 0)),
        ],
        out_shape=[
            jax.ShapeDtypeStruct((B, T, 1536), _BF),
            jax.ShapeDtypeStruct((B, T, 512), _BF),
        ],
    )(x, w1, c1_b, w2, c2_b, w3, c3_b, w4,
      gw, g_b, thw, th_b, phw, ph_b, ww, W_b)

    feat, norms, scores = pl.pallas_call(
        _c_body,
        grid=(nchunks,),
        in_specs=[
            pl.BlockSpec((S, T, 1536), lambda i: (i, 0, 0)),
            pl.BlockSpec((S, T, 512), lambda i: (i, 0, 0)),
            pl.BlockSpec((S, T, F), lambda i: (i, 0, 0)),
            full((3, 1536, F)), full((3, 512, F)), full((F,)),
            full((F, 512)), full((512,)),
            full((512, 128)), full((128,)),
            full((128,)), full((1,)),
        ],
        out_specs=[
            pl.BlockSpec((S, T, F), lambda i: (i, 0, 0)),
            pl.BlockSpec((1, S, T), lambda i: (i, 0, 0)),
            pl.BlockSpec((1, S, T), lambda i: (i, 0, 0)),
        ],
        out_shape=[
            jax.ShapeDtypeStruct((B, T, F), jnp.float32),
            jax.ShapeDtypeStruct((nchunks, S, T), jnp.float32),
            jax.ShapeDtypeStruct((nchunks, S, T), jnp.float32),
        ],
    )(outd, z, x, w5a, w5b, c5_b, f1, fc1_b, f2, fc2_b, f3, fc3_b)
    norms = norms.reshape(B, T)
    scores = scores.reshape(B, T)

    fm, x2, sab, sno, idx2d = pl.pallas_call(
        _d_body,
        grid=(1,),
        in_specs=[full((B, T)), full((B, T))],
        out_specs=[full((NV, T)), full((NV, T)),
                   full((NSIZE, 1)), full((NSIZE, 1)), full((NW, T))],
        out_shape=[
            jax.ShapeDtypeStruct((NV, T), jnp.float32),
            jax.ShapeDtypeStruct((NV, T), jnp.float32),
            jax.ShapeDtypeStruct((NSIZE, 1), jnp.float32),
            jax.ShapeDtypeStruct((NSIZE, 1), jnp.float32),
            jax.ShapeDtypeStruct((NW, T), jnp.int32),
        ],
    )(norms, scores)

    buf = _sc_gather(feat.reshape(B * T, F), idx2d)
    buf = buf.reshape(NW, T, F)[:, :RPW, :].reshape(NW * RPW, F)
    fsa = buf[:480].reshape(160, K, F)
    fsn = buf[480:].reshape(160, K, F)
    xout = x2[:, :, None]
    return (sab, sno, fsa, fsn, fsa, fsa, xout, fsa, fsa, fm)
